# baseline (speedup 1.0000x reference)
import jax
import jax.numpy as jnp
from jax import lax
from jax.experimental import pallas as pl
from jax.experimental.pallas import tpu as pltpu, tpu_sc as plsc

V = 1000000
D = 64
B = 16384
L = 50

NC = 2
NS = 16
LANES = 16
NW = NC * NS
BC = B // NW
NCH = BC // 128
NWAVE = 2
VT = V // 128
TPW = VT // NW + 1

_CP = dict(needs_layout_passes=False, use_tc_tiling_on_sc=True)


def _rsqrt(x):
    i = lax.bitcast_convert_type(x, jnp.int32)
    y = lax.bitcast_convert_type(
        jnp.int32(0x5F3759DF) - lax.shift_right_arithmetic(i, 1), jnp.float32)
    for _ in range(3):
        y = y * (1.5 - 0.5 * x * y * y)
    return y


def _transpose_block(src_v, dst_v, lanes, nv):
    def vstep(v0, _):
        vvec = v0 * LANES + lanes
        lv = lax.shift_right_logical(vvec, 1)
        pof = lax.shift_left(vvec & 1, 6)
        for d0 in range(D):
            dvec = (d0 + lanes) & (D - 1)
            val = plsc.load_gather(src_v, [dvec, vvec])
            plsc.store_scatter(dst_v, [lv, pof + dvec], val)
        return ()

    lax.fori_loop(0, nv, vstep, (), unroll=False)


def _k1_body(win_t, wout_t, tin2, tout2, comb,
             buf_a, buf_b, line_a, line_b, sem_r, sem_w):
    wid = lax.axis_index("s") * NC + lax.axis_index("c")
    lanes = lax.iota(jnp.int32, LANES)

    for tab, w_t, tl2 in ((0, win_t, tin2), (1, wout_t, tout2)):
        obase = tab * (V // 2)

        def read(t, buf):
            pltpu.async_copy(w_t.at[:, pl.ds(t * 128, 128)], buf, sem_r)

        def wait_read(t, buf):
            pltpu.make_async_copy(w_t.at[:, pl.ds(t * 128, 128)], buf,
                                  sem_r).wait()

        def write(t, line):
            pltpu.async_copy(line, comb.at[pl.ds(obase + t * 64, 64), :],
                             sem_w)

        def wait_write(t, line):
            pltpu.make_async_copy(line,
                                  comb.at[pl.ds(obase + t * 64, 64), :],
                                  sem_w).wait()

        t0 = wid
        read(t0, buf_a)

        def step(i, _):
            ta = t0 + 2 * i * NW
            tb = ta + NW

            @pl.when(ta < VT)
            def _():
                wait_read(ta, buf_a)

                @pl.when(tb < VT)
                def _():
                    read(tb, buf_b)
                _transpose_block(buf_a, line_a, lanes, 8)
                write(ta, line_a)

                @pl.when(tb < VT)
                def _():
                    wait_read(tb, buf_b)
                    tn = ta + 2 * NW

                    @pl.when(tn < VT)
                    def _():
                        read(tn, buf_a)
                    _transpose_block(buf_b, line_b, lanes, 8)
                    write(tb, line_b)
                wait_write(ta, line_a)

                @pl.when(tb < VT)
                def _():
                    wait_write(tb, line_b)
            return ()

        lax.fori_loop(0, (TPW + 1) // 2, step, (), unroll=False)

        @pl.when(wid == 0)
        def _():
            pltpu.sync_copy(tl2, line_a.at[pl.ds(0, 32), :])
            pltpu.sync_copy(line_a.at[pl.ds(0, 32), :],
                            comb.at[pl.ds(obase + VT * 64, 32), :])


def _k2_body(cen_hbm, ctx_hbm, comb, out_hbm,
             ridx_v, hidx_v, poff_v, in_v, wave_v, invin_v, res_v, sem):
    wid = lax.axis_index("s") * NC + lax.axis_index("c")
    base = wid * BC
    lanes = lax.iota(jnp.int32, LANES)

    def halve_indices(off):
        for j in range(NCH):
            for k in range(8):
                v = ridx_v[j, pl.ds(k * LANES, LANES)]
                hidx_v[j, pl.ds(k * LANES, LANES)] = (
                    lax.shift_right_logical(v, 1) + off)
                poff_v[pl.ds((j * 8 + k) * LANES, LANES)] = lax.shift_left(
                    v & 1, 6)

    for j in range(NCH):
        pltpu.sync_copy(cen_hbm.at[pl.ds(base + j * 128, 128)], ridx_v.at[j])
    halve_indices(0)
    for w in range(NCH // 2):
        for j in range(2):
            pltpu.async_copy(comb.at[hidx_v.at[w * 2 + j]],
                             wave_v.at[pl.ds(j * 128, 128), :], sem)
        for j in range(2):
            pltpu.make_async_copy(comb.at[hidx_v.at[w * 2 + j]],
                                  wave_v.at[pl.ds(j * 128, 128), :], sem).wait()

        def cgrp(g, _):
            rows = g * LANES + lanes
            gpos = w * 256 + g * LANES + lanes
            po = plsc.load_gather(poff_v, [gpos])
            acc = jnp.zeros((LANES,), jnp.float32)
            for d in range(D):
                col = (lanes + d) & (D - 1)
                v = plsc.load_gather(wave_v, [rows, col + po])
                plsc.store_scatter(in_v, [gpos, col], v)
                acc += v * v
            invin_v[pl.ds(w * 256 + g * LANES, LANES)] = _rsqrt(acc)
            return ()

        lax.fori_loop(0, 256 // LANES, cgrp, (), unroll=False)

    def l_body(l, _):
        for j in range(NCH):
            pltpu.sync_copy(ctx_hbm.at[l, pl.ds(base + j * 128, 128)],
                            ridx_v.at[j])
        halve_indices(V // 2)

        for w in range(NWAVE):
            for j in range(2):
                pltpu.async_copy(comb.at[hidx_v.at[w * 2 + j]],
                                 wave_v.at[pl.ds(j * 128, 128), :], sem)
            for j in range(2):
                pltpu.make_async_copy(comb.at[hidx_v.at[w * 2 + j]],
                                      wave_v.at[pl.ds(j * 128, 128), :],
                                      sem).wait()

            def g_body(g, _):
                rows = g * LANES + lanes
                gpos = w * 256 + g * LANES + lanes
                po = plsc.load_gather(poff_v, [gpos])
                acc_d = jnp.zeros((LANES,), jnp.float32)
                acc_s = jnp.zeros((LANES,), jnp.float32)
                for d in range(D):
                    col = (lanes + d) & (D - 1)
                    o = plsc.load_gather(wave_v, [rows, col + po])
                    i = plsc.load_gather(in_v, [gpos, col])
                    acc_d += o * i
                    acc_s += o * o
                res = (acc_d * _rsqrt(acc_s)
                       * invin_v[pl.ds(w * 256 + g * LANES, LANES)])
                res_v[pl.ds(w * 256 + g * LANES, LANES)] = res
                return ()

            lax.fori_loop(0, 256 // LANES, g_body, (), unroll=False)

        pltpu.sync_copy(res_v, out_hbm.at[l, pl.ds(base, BC)])
        return ()

    lax.fori_loop(0, L, l_body, (), unroll=False)


@jax.jit
def kernel(center, context, emb_in_weight, emb_out_weight):
    mesh = plsc.VectorSubcoreMesh(core_axis_name="c", subcore_axis_name="s")

    k1 = pl.kernel(
        _k1_body,
        out_type=jax.ShapeDtypeStruct((V, 2 * D), jnp.float32),
        mesh=mesh,
        compiler_params=pltpu.CompilerParams(**_CP),
        scratch_types=[
            pltpu.VMEM((D, 128), jnp.float32),
            pltpu.VMEM((D, 128), jnp.float32),
            pltpu.VMEM((D, 128), jnp.float32),
            pltpu.VMEM((D, 128), jnp.float32),
            pltpu.SemaphoreType.DMA,
            pltpu.SemaphoreType.DMA,
        ],
    )
    tin2 = emb_in_weight[VT * 128:, :].reshape(32, 2 * D)
    tout2 = emb_out_weight[VT * 128:, :].reshape(32, 2 * D)
    comb = k1(emb_in_weight.T, emb_out_weight.T, tin2, tout2)

    k2 = pl.kernel(
        _k2_body,
        out_type=jax.ShapeDtypeStruct((L, B), jnp.float32),
        mesh=mesh,
        compiler_params=pltpu.CompilerParams(**_CP),
        scratch_types=[
            pltpu.VMEM((NCH, 128), jnp.int32),
            pltpu.VMEM((NCH, 128), jnp.int32),
            pltpu.VMEM((BC,), jnp.int32),
            pltpu.VMEM((BC, D), jnp.float32),
            pltpu.VMEM((256, 2 * D), jnp.float32),
            pltpu.VMEM((BC,), jnp.float32),
            pltpu.VMEM((BC,), jnp.float32),
            pltpu.SemaphoreType.DMA,
        ],
    )
    return k2(center, context, comb)


# speedup vs baseline: 1.0795x; 1.0795x over previous
import jax
import jax.numpy as jnp
from jax import lax
from jax.experimental import pallas as pl
from jax.experimental.pallas import tpu as pltpu, tpu_sc as plsc

V = 1000000
D = 64
B = 16384
L = 50

NC = 2
NS = 16
LANES = 16
NW = NC * NS
BC = B // NW
NCH = BC // 128
NWAVE = 2
VT = V // 128
TPW = VT // NW + 1

_CP = dict(needs_layout_passes=False, use_tc_tiling_on_sc=True)


def _rsqrt(x):
    i = lax.bitcast_convert_type(x, jnp.int32)
    y = lax.bitcast_convert_type(
        jnp.int32(0x5F3759DF) - lax.shift_right_arithmetic(i, 1), jnp.float32)
    for _ in range(3):
        y = y * (1.5 - 0.5 * x * y * y)
    return y


def _transpose_block(src_v, dst_v, lanes, nv):
    @plsc.parallel_loop(0, nv)
    def vstep(v0):
        vvec = v0 * LANES + lanes
        lv = lax.shift_right_logical(vvec, 1)
        pof = lax.shift_left(vvec & 1, 6)
        for d0 in range(D):
            dvec = (d0 + lanes) & (D - 1)
            val = plsc.load_gather(src_v, [dvec, vvec])
            plsc.store_scatter(dst_v, [lv, pof + dvec], val)


def _k1_body(win_t, wout_t, tin2, tout2, comb,
             buf_a, buf_b, line_a, line_b, sem_r, sem_w):
    wid = lax.axis_index("s") * NC + lax.axis_index("c")
    lanes = lax.iota(jnp.int32, LANES)

    for tab, w_t, tl2 in ((0, win_t, tin2), (1, wout_t, tout2)):
        obase = tab * (V // 2)

        def read(t, buf):
            pltpu.async_copy(w_t.at[:, pl.ds(t * 128, 128)], buf, sem_r)

        def wait_read(t, buf):
            pltpu.make_async_copy(w_t.at[:, pl.ds(t * 128, 128)], buf,
                                  sem_r).wait()

        def write(t, line):
            pltpu.async_copy(line, comb.at[pl.ds(obase + t * 64, 64), :],
                             sem_w)

        def wait_write(t, line):
            pltpu.make_async_copy(line,
                                  comb.at[pl.ds(obase + t * 64, 64), :],
                                  sem_w).wait()

        t0 = wid
        read(t0, buf_a)

        def step(i, _):
            ta = t0 + 2 * i * NW
            tb = ta + NW

            @pl.when(ta < VT)
            def _():
                wait_read(ta, buf_a)

                @pl.when(tb < VT)
                def _():
                    read(tb, buf_b)
                _transpose_block(buf_a, line_a, lanes, 8)
                write(ta, line_a)

                @pl.when(tb < VT)
                def _():
                    wait_read(tb, buf_b)
                    tn = ta + 2 * NW

                    @pl.when(tn < VT)
                    def _():
                        read(tn, buf_a)
                    _transpose_block(buf_b, line_b, lanes, 8)
                    write(tb, line_b)
                wait_write(ta, line_a)

                @pl.when(tb < VT)
                def _():
                    wait_write(tb, line_b)
            return ()

        lax.fori_loop(0, (TPW + 1) // 2, step, (), unroll=False)

        @pl.when(wid == 0)
        def _():
            pltpu.sync_copy(tl2, line_a.at[pl.ds(0, 32), :])
            pltpu.sync_copy(line_a.at[pl.ds(0, 32), :],
                            comb.at[pl.ds(obase + VT * 64, 32), :])


def _k2_body(cen_hbm, ctx_hbm, comb, out_hbm,
             ridx_v, hidx_v, poff_v, in_v, wave_v, invin_v, res_v, sem):
    wid = lax.axis_index("s") * NC + lax.axis_index("c")
    base = wid * BC
    lanes = lax.iota(jnp.int32, LANES)

    def halve_indices(off):
        for j in range(NCH):
            for k in range(8):
                v = ridx_v[j, pl.ds(k * LANES, LANES)]
                hidx_v[j, pl.ds(k * LANES, LANES)] = (
                    lax.shift_right_logical(v, 1) + off)
                poff_v[pl.ds((j * 8 + k) * LANES, LANES)] = lax.shift_left(
                    v & 1, 6)

    for j in range(NCH):
        pltpu.sync_copy(cen_hbm.at[pl.ds(base + j * 128, 128)], ridx_v.at[j])
    halve_indices(0)
    for w in range(NCH // 2):
        for j in range(2):
            pltpu.async_copy(comb.at[hidx_v.at[w * 2 + j]],
                             wave_v.at[pl.ds(j * 128, 128), :], sem)
        for j in range(2):
            pltpu.make_async_copy(comb.at[hidx_v.at[w * 2 + j]],
                                  wave_v.at[pl.ds(j * 128, 128), :], sem).wait()

        @plsc.parallel_loop(0, 256 // LANES)
        def cgrp(g):
            rows = g * LANES + lanes
            gpos = w * 256 + g * LANES + lanes
            po = plsc.load_gather(poff_v, [gpos])
            acc = jnp.zeros((LANES,), jnp.float32)
            for d in range(D):
                col = (lanes + d) & (D - 1)
                v = plsc.load_gather(wave_v, [rows, col + po])
                plsc.store_scatter(in_v, [gpos, col], v)
                acc += v * v
            invin_v[pl.ds(w * 256 + g * LANES, LANES)] = _rsqrt(acc)

    def l_body(l, _):
        for j in range(NCH):
            pltpu.sync_copy(ctx_hbm.at[l, pl.ds(base + j * 128, 128)],
                            ridx_v.at[j])
        halve_indices(V // 2)

        for w in range(NWAVE):
            for j in range(2):
                pltpu.async_copy(comb.at[hidx_v.at[w * 2 + j]],
                                 wave_v.at[pl.ds(j * 128, 128), :], sem)
            for j in range(2):
                pltpu.make_async_copy(comb.at[hidx_v.at[w * 2 + j]],
                                      wave_v.at[pl.ds(j * 128, 128), :],
                                      sem).wait()

            @plsc.parallel_loop(0, 256 // LANES)
            def g_body(g):
                rows = g * LANES + lanes
                gpos = w * 256 + g * LANES + lanes
                po = plsc.load_gather(poff_v, [gpos])
                acc_d = jnp.zeros((LANES,), jnp.float32)
                acc_s = jnp.zeros((LANES,), jnp.float32)
                for d in range(D):
                    col = (lanes + d) & (D - 1)
                    o = plsc.load_gather(wave_v, [rows, col + po])
                    i = plsc.load_gather(in_v, [gpos, col])
                    acc_d += o * i
                    acc_s += o * o
                res = (acc_d * _rsqrt(acc_s)
                       * invin_v[pl.ds(w * 256 + g * LANES, LANES)])
                res_v[pl.ds(w * 256 + g * LANES, LANES)] = res

        pltpu.sync_copy(res_v, out_hbm.at[l, pl.ds(base, BC)])
        return ()

    lax.fori_loop(0, L, l_body, (), unroll=False)


@jax.jit
def kernel(center, context, emb_in_weight, emb_out_weight):
    mesh = plsc.VectorSubcoreMesh(core_axis_name="c", subcore_axis_name="s")

    k1 = pl.kernel(
        _k1_body,
        out_type=jax.ShapeDtypeStruct((V, 2 * D), jnp.float32),
        mesh=mesh,
        compiler_params=pltpu.CompilerParams(**_CP),
        scratch_types=[
            pltpu.VMEM((D, 128), jnp.float32),
            pltpu.VMEM((D, 128), jnp.float32),
            pltpu.VMEM((D, 128), jnp.float32),
            pltpu.VMEM((D, 128), jnp.float32),
            pltpu.SemaphoreType.DMA,
            pltpu.SemaphoreType.DMA,
        ],
    )
    tin2 = emb_in_weight[VT * 128:, :].reshape(32, 2 * D)
    tout2 = emb_out_weight[VT * 128:, :].reshape(32, 2 * D)
    comb = k1(emb_in_weight.T, emb_out_weight.T, tin2, tout2)

    k2 = pl.kernel(
        _k2_body,
        out_type=jax.ShapeDtypeStruct((L, B), jnp.float32),
        mesh=mesh,
        compiler_params=pltpu.CompilerParams(**_CP),
        scratch_types=[
            pltpu.VMEM((NCH, 128), jnp.int32),
            pltpu.VMEM((NCH, 128), jnp.int32),
            pltpu.VMEM((BC,), jnp.int32),
            pltpu.VMEM((BC, D), jnp.float32),
            pltpu.VMEM((256, 2 * D), jnp.float32),
            pltpu.VMEM((BC,), jnp.float32),
            pltpu.VMEM((BC,), jnp.float32),
            pltpu.SemaphoreType.DMA,
        ],
    )
    return k2(center, context, comb)
